# baseline (speedup 1.0000x reference)
import functools

import jax
import jax.numpy as jnp
from jax import lax
from jax.experimental import pallas as pl
from jax.experimental.pallas import tpu as pltpu
from jax.experimental.pallas import tpu_sc as plsc

_NC = 2
_NS = 16
_NW = _NC * _NS

_N = 10000
_E = 160000
_D = 128

_K = 128
_CHUNKS = 40
_EPT = _CHUNKS * _K
_ERPT = _E // _NW
_EPAD = _EPT - _ERPT
_EP = _NW * _EPT

_NP = 10240
_ROWS_PT = _NP // _NS
_RCHUNK = 128
_RSTEPS = _ROWS_PT // _RCHUNK

_SC_MESH = dict(core_axis_name="c", subcore_axis_name="s")
_SC_PARAMS = pltpu.CompilerParams(use_tc_tiling_on_sc=False)


_DUMP = _N
_G = 2
_NGRP = _CHUNKS // _G


def _segsum_sc(width, with_deg=False):
    mesh = plsc.VectorSubcoreMesh(**_SC_MESH)

    out_type = [jax.ShapeDtypeStruct((_NC, _NP, width), jnp.float32)]
    scratch = [
        pltpu.VMEM((_CHUNKS, _K), jnp.int32),
        pltpu.VMEM((_CHUNKS, _K), jnp.int32),
        pltpu.VMEM((1, _K, width), jnp.float32),
        pltpu.VMEM((_RCHUNK, width), jnp.float32),
        pltpu.VMEM_SHARED((_NP, width), jnp.float32),
        pltpu.SemaphoreType.DMA,
        pltpu.SemaphoreType.DMA,
    ]
    if with_deg:
        out_type.append(jax.ShapeDtypeStruct((_NC, _NP), jnp.float32))
        scratch += [
            pltpu.VMEM((_K,), jnp.float32),
            pltpu.VMEM((_ROWS_PT,), jnp.float32),
            pltpu.VMEM_SHARED((_NP,), jnp.float32),
        ]

    @functools.partial(
        pl.kernel,
        out_type=tuple(out_type),
        mesh=mesh,
        compiler_params=_SC_PARAMS,
        scratch_types=scratch,
    )
    def seg(table, src3, dst3, zeros, *rest):
        if with_deg:
            (zeros1, ones1, out, out2, srcv, dstv, buf, stage, acc,
             gsem, ssem, onesv, stage1, accdeg) = rest
        else:
            out, srcv, dstv, buf, stage, acc, gsem, ssem = rest
        cid = lax.axis_index("c")
        sid = lax.axis_index("s")
        wid = cid * _NS + sid
        pltpu.sync_copy(src3.at[wid], srcv)
        pltpu.sync_copy(dst3.at[wid], dstv)
        pltpu.sync_copy(zeros, stage)
        row0 = sid * _ROWS_PT
        for j in range(_RSTEPS):
            pltpu.sync_copy(stage, acc.at[pl.ds(row0 + j * _RCHUNK, _RCHUNK)])
        if with_deg:
            pltpu.sync_copy(ones1, onesv)
            pltpu.sync_copy(zeros1, stage1)
            pltpu.sync_copy(stage1, accdeg.at[pl.ds(row0, _ROWS_PT)])
        plsc.subcore_barrier()

        def chunk(i, carry):
            pltpu.sync_copy(table.at[srcv.at[i]], buf.at[0])
            d = pltpu.async_copy(buf.at[0], acc.at[dstv.at[i]], ssem,
                                 add=True)
            if with_deg:
                d2 = pltpu.async_copy(onesv, accdeg.at[dstv.at[i]], ssem,
                                      add=True)
                d2.wait()
            d.wait()
            return carry

        lax.fori_loop(0, _CHUNKS, chunk, 0)
        plsc.subcore_barrier()
        for j in range(_RSTEPS):
            r = row0 + j * _RCHUNK
            pltpu.sync_copy(acc.at[pl.ds(r, _RCHUNK)], stage)
            pltpu.sync_copy(stage, out.at[cid, pl.ds(r, _RCHUNK)])
        if with_deg:
            pltpu.sync_copy(accdeg.at[pl.ds(row0, _ROWS_PT)], stage1)
            pltpu.sync_copy(stage1, out2.at[cid, pl.ds(row0, _ROWS_PT)])

    return seg


_K2 = 64
_C2 = _EPT // _K2


def _edge_gather_sc():
    mesh = plsc.VectorSubcoreMesh(**_SC_MESH)
    w = 2 * _D

    @functools.partial(
        pl.kernel,
        out_type=(
            jax.ShapeDtypeStruct((_EP, w), jnp.float32),
            jax.ShapeDtypeStruct((_EP, w), jnp.float32),
        ),
        mesh=mesh,
        compiler_params=_SC_PARAMS,
        scratch_types=[
            pltpu.VMEM((_CHUNKS, _K), jnp.int32),
            pltpu.VMEM((_CHUNKS, _K), jnp.int32),
            pltpu.VMEM((_K, w), jnp.float32),
            pltpu.VMEM((_K, w), jnp.float32),
            pltpu.SemaphoreType.DMA,
        ],
    )
    def gat(srctab, dsttab, src3, dst3, gs, gd, srcv, dstv, bufa, bufb,
            wsem):
        cid = lax.axis_index("c")
        sid = lax.axis_index("s")
        wid = cid * _NS + sid
        pltpu.sync_copy(src3.at[wid], srcv)
        pltpu.sync_copy(dst3.at[wid], dstv)
        base = wid * _EPT

        def chunk(i, carry):
            e0 = base + i * _K
            pltpu.sync_copy(srctab.at[srcv.at[i]], bufa)
            wa = pltpu.async_copy(bufa, gs.at[pl.ds(e0, _K)], wsem)
            pltpu.sync_copy(dsttab.at[dstv.at[i]], bufb)
            wb = pltpu.async_copy(bufb, gd.at[pl.ds(e0, _K)], wsem)
            wa.wait()
            wb.wait()
            return carry

        lax.fori_loop(0, _CHUNKS, chunk, 0)

    return gat


_BN = 1000


def _tc_layer1(P, degP, x, W1s, W1n, b1):
    def body(p_ref, deg_ref, x_ref, ws_ref, wn_ref, b_ref, h_ref, dinv_ref):
        p = p_ref[0] + p_ref[1]
        dinv = 1.0 / jnp.maximum(deg_ref[0] + deg_ref[1], 1.0)
        mean = p * dinv
        h = (jnp.dot(x_ref[...], ws_ref[...], preferred_element_type=jnp.float32)
             + jnp.dot(mean, wn_ref[...], preferred_element_type=jnp.float32)
             + b_ref[...])
        h_ref[...] = jnp.maximum(h, 0.0)
        dinv_ref[...] = dinv

    return pl.pallas_call(
        body,
        grid=(_N // _BN,),
        in_specs=[
            pl.BlockSpec((_NC, _BN, _D), lambda i: (0, i, 0)),
            pl.BlockSpec((_NC, _BN, 1), lambda i: (0, i, 0)),
            pl.BlockSpec((_BN, _D), lambda i: (i, 0)),
            pl.BlockSpec((_D, _D), lambda i: (0, 0)),
            pl.BlockSpec((_D, _D), lambda i: (0, 0)),
            pl.BlockSpec((_D,), lambda i: (0,)),
        ],
        out_specs=[
            pl.BlockSpec((_BN, _D), lambda i: (i, 0)),
            pl.BlockSpec((_BN, 1), lambda i: (i, 0)),
        ],
        out_shape=[
            jax.ShapeDtypeStruct((_N, _D), jnp.float32),
            jax.ShapeDtypeStruct((_N, 1), jnp.float32),
        ],
    )(P, degP, x, W1s, W1n, b1)


def _tc_layer2_tables(M, h1, dinv, W2s, W2n, b2, Wn, bn, We1, be1, Wl1):
    def body(m_ref, h1_ref, dinv_ref, w2s, w2n, b2r, wnr, bnr, we1, be1r, wl1,
             srct_ref, dstt_ref):
        m = m_ref[0] + m_ref[1]
        mean2 = m * dinv_ref[...]
        h1b = h1_ref[...]
        h2 = (jnp.dot(h1b, w2s[...], preferred_element_type=jnp.float32)
              + jnp.dot(mean2, w2n[...], preferred_element_type=jnp.float32)
              + b2r[...])
        h2 = jnp.maximum(h2, 0.0)
        hn = jnp.maximum(
            jnp.dot(h2, wnr[...], preferred_element_type=jnp.float32) + bnr[...],
            0.0)
        we1v = we1[...]
        wl1v = wl1[...]
        a = jnp.dot(hn, we1v[0:_D], preferred_element_type=jnp.float32) + be1r[...]
        c = jnp.dot(hn, wl1v[0:_D], preferred_element_type=jnp.float32)
        b = jnp.dot(hn, we1v[_D:2 * _D], preferred_element_type=jnp.float32)
        d = jnp.dot(hn, wl1v[_D:2 * _D], preferred_element_type=jnp.float32)
        srct_ref[...] = jnp.concatenate([a, c], axis=1)
        dstt_ref[...] = jnp.concatenate([b, d], axis=1)

    full = lambda shape: pl.BlockSpec(shape, lambda i: tuple(0 for _ in shape))
    return pl.pallas_call(
        body,
        grid=(_N // _BN,),
        in_specs=[
            pl.BlockSpec((_NC, _BN, _D), lambda i: (0, i, 0)),
            pl.BlockSpec((_BN, _D), lambda i: (i, 0)),
            pl.BlockSpec((_BN, 1), lambda i: (i, 0)),
            full((_D, _D)),
            full((_D, _D)),
            full((_D,)),
            full((_D, _D)),
            full((_D,)),
            full((2 * _D + 5, _D)),
            full((_D,)),
            full((2 * _D + 5, _D)),
        ],
        out_specs=[
            pl.BlockSpec((_BN, 2 * _D), lambda i: (i, 0)),
            pl.BlockSpec((_BN, 2 * _D), lambda i: (i, 0)),
        ],
        out_shape=[
            jax.ShapeDtypeStruct((_N, 2 * _D), jnp.float32),
            jax.ShapeDtypeStruct((_N, 2 * _D), jnp.float32),
        ],
    )(M, h1, dinv, W2s, W2n, b2, Wn, bn, We1, be1, Wl1)


_BE = 2048


def _tc_edge_mlp(Gs, Gd, ea, We1, We2, be2, Wl1, bl1, Wl2, bl2):
    def body(gs_ref, gd_ref, ea_ref, we1, we2, be2r, wl1, bl1r, wl2, bl2r,
             o_ref):
        gs = gs_ref[...]
        gd = gd_ref[...]
        eab = ea_ref[...]
        we1c = we1[...][2 * _D:]
        wl1c = wl1[...][2 * _D:]
        q = (gs[:, :_D] + gd[:, :_D]
             + jnp.dot(eab, we1c, preferred_element_type=jnp.float32))
        g = jnp.tanh(q)
        s = jnp.dot(g, we2[...], preferred_element_type=jnp.float32) + be2r[...]
        w = jax.nn.sigmoid(s)
        p = (gs[:, _D:] + gd[:, _D:]
             + jnp.dot(eab, wl1c, preferred_element_type=jnp.float32))
        t = jnp.maximum(p * w + bl1r[...], 0.0)
        o = jnp.dot(t, wl2[...], preferred_element_type=jnp.float32) + bl2r[...]
        o_ref[...] = o

    full = lambda shape: pl.BlockSpec(shape, lambda i: tuple(0 for _ in shape))
    return pl.pallas_call(
        body,
        grid=(_EP // _BE,),
        in_specs=[
            pl.BlockSpec((_BE, 2 * _D), lambda i: (i, 0)),
            pl.BlockSpec((_BE, 2 * _D), lambda i: (i, 0)),
            pl.BlockSpec((_BE, 5), lambda i: (i, 0)),
            full((2 * _D + 5, _D)),
            full((_D, 1)),
            full((1,)),
            full((2 * _D + 5, _D)),
            full((_D,)),
            full((_D, 1)),
            full((1,)),
        ],
        out_specs=pl.BlockSpec((_BE, 1), lambda i: (i, 0)),
        out_shape=jax.ShapeDtypeStruct((_EP, 1), jnp.float32),
    )(Gs, Gd, ea, We1, We2, be2, Wl1, bl1, Wl2, bl2)


def kernel(x, edge_index, edge_attr, W1_self, W1_neigh, b1, W2_self, W2_neigh,
           b2, Wn, bn, We1, be1, We2, be2, Wl1, bl1, Wl2, bl2):
    srcm = edge_index[0].reshape(_NW, _ERPT)
    dstm = edge_index[1].reshape(_NW, _ERPT)
    src3 = jnp.pad(srcm, ((0, 0), (0, _EPAD))).reshape(_NW, _CHUNKS, _K)
    dst3 = jnp.pad(dstm, ((0, 0), (0, _EPAD)),
                   constant_values=_DUMP).reshape(_NW, _CHUNKS, _K)
    ea_p = jnp.pad(edge_attr.reshape(_NW, _ERPT, 5),
                   ((0, 0), (0, _EPAD), (0, 0))).reshape(_EP, 5)
    src3e = src3.reshape(_NW, _C2, _K2)
    dst3e = dst3.reshape(_NW, _C2, _K2)
    zeros_d = jnp.zeros((_RCHUNK, _D), jnp.float32)
    zeros1 = jnp.zeros((_ROWS_PT,), jnp.float32)
    ones1 = jnp.ones((_K,), jnp.float32)

    P, degPf = _segsum_sc(_D, with_deg=True)(x, src3, dst3, zeros_d,
                                             zeros1, ones1)
    P = P[:, :_N]
    degP = degPf[:, :_N].reshape(_NC, _N, 1)
    h1, dinv = _tc_layer1(P, degP, x, W1_self, W1_neigh, b1)
    (M,) = _segsum_sc(_D)(h1, src3, dst3, zeros_d)
    M = M[:, :_N]
    srctab, dsttab = _tc_layer2_tables(M, h1, dinv, W2_self, W2_neigh, b2,
                                       Wn, bn, We1, be1, Wl1)
    gs, gd = _edge_gather_sc()(srctab, dsttab, src3, dst3)
    o = _tc_edge_mlp(gs, gd, ea_p, We1, We2, be2, Wl1, bl1, Wl2, bl2)
    return o.reshape(_NW, _EPT)[:, :_ERPT].reshape(_E, 1)


# speedup vs baseline: 1.4845x; 1.4845x over previous
import functools

import jax
import jax.numpy as jnp
from jax import lax
from jax.experimental import pallas as pl
from jax.experimental.pallas import tpu as pltpu
from jax.experimental.pallas import tpu_sc as plsc

_NC = 2
_NS = 16
_NW = _NC * _NS

_N = 10000
_E = 160000
_D = 128

_K = 125
_CHUNKS = 40
_EPT = _CHUNKS * _K
_ERPT = _E // _NW
_EPAD = _EPT - _ERPT
_EP = _NW * _EPT

_NP = 10240
_ROWS_PT = _NP // _NS
_RCHUNK = 128
_RSTEPS = _ROWS_PT // _RCHUNK

_SC_MESH = dict(core_axis_name="c", subcore_axis_name="s")
_SC_PARAMS = pltpu.CompilerParams(use_tc_tiling_on_sc=False)


_DUMP = _N
_G = 2
_NGRP = _CHUNKS // _G


def _segsum_sc(width, with_deg=False):
    mesh = plsc.VectorSubcoreMesh(**_SC_MESH)

    out_type = [jax.ShapeDtypeStruct((_NC, _NP, width), jnp.float32)]
    scratch = [
        pltpu.VMEM((_CHUNKS, _K), jnp.int32),
        pltpu.VMEM((_CHUNKS, _K), jnp.int32),
        pltpu.VMEM((_K, width), jnp.float32),
        pltpu.VMEM((_RCHUNK, width), jnp.float32),
        pltpu.VMEM_SHARED((_NP, width), jnp.float32),
        pltpu.SemaphoreType.DMA,
        pltpu.SemaphoreType.DMA,
    ]
    if with_deg:
        out_type.append(jax.ShapeDtypeStruct((_NC, _NP), jnp.float32))
        scratch += [
            pltpu.VMEM((_K,), jnp.float32),
            pltpu.VMEM((_ROWS_PT,), jnp.float32),
            pltpu.VMEM_SHARED((_NP,), jnp.float32),
        ]

    @functools.partial(
        pl.kernel,
        out_type=tuple(out_type),
        mesh=mesh,
        compiler_params=_SC_PARAMS,
        scratch_types=scratch,
    )
    def seg(table, src3, dst3, zeros, *rest):
        if with_deg:
            (zeros1, ones1, out, out2, srcv, dstv, buf, stage, acc,
             gsem, ssem, onesv, stage1, accdeg) = rest
        else:
            out, srcv, dstv, buf, stage, acc, gsem, ssem = rest
        cid = lax.axis_index("c")
        sid = lax.axis_index("s")
        wid = cid * _NS + sid
        pltpu.sync_copy(src3.at[wid], srcv)
        pltpu.sync_copy(dst3.at[wid], dstv)
        pltpu.sync_copy(zeros, stage)
        row0 = sid * _ROWS_PT
        for j in range(_RSTEPS):
            pltpu.sync_copy(stage, acc.at[pl.ds(row0 + j * _RCHUNK, _RCHUNK)])
        if with_deg:
            pltpu.sync_copy(ones1, onesv)
            pltpu.sync_copy(zeros1, stage1)
            pltpu.sync_copy(stage1, accdeg.at[pl.ds(row0, _ROWS_PT)])
        plsc.subcore_barrier()

        def chunk(i, carry):
            pltpu.sync_copy(table.at[srcv.at[i]], buf)
            pltpu.sync_copy(buf, acc.at[dstv.at[i]], add=True)
            if with_deg:
                pltpu.sync_copy(onesv, accdeg.at[dstv.at[i]], add=True)
            return carry

        lax.fori_loop(0, _CHUNKS, chunk, 0)
        plsc.subcore_barrier()
        for j in range(_RSTEPS):
            r = row0 + j * _RCHUNK
            pltpu.sync_copy(acc.at[pl.ds(r, _RCHUNK)], stage)
            pltpu.sync_copy(stage, out.at[cid, pl.ds(r, _RCHUNK)])
        if with_deg:
            pltpu.sync_copy(accdeg.at[pl.ds(row0, _ROWS_PT)], stage1)
            pltpu.sync_copy(stage1, out2.at[cid, pl.ds(row0, _ROWS_PT)])

    return seg


_K2 = 64
_C2 = _EPT // _K2


def _edge_gather_sc():
    mesh = plsc.VectorSubcoreMesh(**_SC_MESH)
    w = 2 * _D

    @functools.partial(
        pl.kernel,
        out_type=(
            jax.ShapeDtypeStruct((_EP, w), jnp.float32),
            jax.ShapeDtypeStruct((_EP, w), jnp.float32),
        ),
        mesh=mesh,
        compiler_params=_SC_PARAMS,
        scratch_types=[
            pltpu.VMEM((_CHUNKS, _K), jnp.int32),
            pltpu.VMEM((_CHUNKS, _K), jnp.int32),
            pltpu.VMEM((_K, w), jnp.float32),
            pltpu.VMEM((_K, w), jnp.float32),
            pltpu.SemaphoreType.DMA,
        ],
    )
    def gat(srctab, dsttab, src3, dst3, gs, gd, srcv, dstv, bufa, bufb,
            wsem):
        cid = lax.axis_index("c")
        sid = lax.axis_index("s")
        wid = cid * _NS + sid
        pltpu.sync_copy(src3.at[wid], srcv)
        pltpu.sync_copy(dst3.at[wid], dstv)
        base = wid * _EPT

        def chunk(i, carry):
            e0 = base + i * _K
            pltpu.sync_copy(srctab.at[srcv.at[i]], bufa)
            pltpu.sync_copy(bufa, gs.at[pl.ds(e0, _K)])
            pltpu.sync_copy(dsttab.at[dstv.at[i]], bufb)
            pltpu.sync_copy(bufb, gd.at[pl.ds(e0, _K)])
            return carry

        lax.fori_loop(0, _CHUNKS, chunk, 0)

    return gat


_BN = 1000


def _tc_layer1(P, degP, x, W1s, W1n, b1):
    def body(p_ref, deg_ref, x_ref, ws_ref, wn_ref, b_ref, h_ref, dinv_ref):
        p = p_ref[0] + p_ref[1]
        dinv = 1.0 / jnp.maximum(deg_ref[0] + deg_ref[1], 1.0)
        mean = p * dinv
        h = (jnp.dot(x_ref[...], ws_ref[...], preferred_element_type=jnp.float32)
             + jnp.dot(mean, wn_ref[...], preferred_element_type=jnp.float32)
             + b_ref[...])
        h_ref[...] = jnp.maximum(h, 0.0)
        dinv_ref[...] = dinv

    return pl.pallas_call(
        body,
        grid=(_N // _BN,),
        in_specs=[
            pl.BlockSpec((_NC, _BN, _D), lambda i: (0, i, 0)),
            pl.BlockSpec((_NC, _BN, 1), lambda i: (0, i, 0)),
            pl.BlockSpec((_BN, _D), lambda i: (i, 0)),
            pl.BlockSpec((_D, _D), lambda i: (0, 0)),
            pl.BlockSpec((_D, _D), lambda i: (0, 0)),
            pl.BlockSpec((_D,), lambda i: (0,)),
        ],
        out_specs=[
            pl.BlockSpec((_BN, _D), lambda i: (i, 0)),
            pl.BlockSpec((_BN, 1), lambda i: (i, 0)),
        ],
        out_shape=[
            jax.ShapeDtypeStruct((_N, _D), jnp.float32),
            jax.ShapeDtypeStruct((_N, 1), jnp.float32),
        ],
    )(P, degP, x, W1s, W1n, b1)


def _tc_layer2_tables(M, h1, dinv, W2s, W2n, b2, Wn, bn, We1, be1, Wl1):
    def body(m_ref, h1_ref, dinv_ref, w2s, w2n, b2r, wnr, bnr, we1, be1r, wl1,
             srct_ref, dstt_ref):
        m = m_ref[0] + m_ref[1]
        mean2 = m * dinv_ref[...]
        h1b = h1_ref[...]
        h2 = (jnp.dot(h1b, w2s[...], preferred_element_type=jnp.float32)
              + jnp.dot(mean2, w2n[...], preferred_element_type=jnp.float32)
              + b2r[...])
        h2 = jnp.maximum(h2, 0.0)
        hn = jnp.maximum(
            jnp.dot(h2, wnr[...], preferred_element_type=jnp.float32) + bnr[...],
            0.0)
        we1v = we1[...]
        wl1v = wl1[...]
        a = jnp.dot(hn, we1v[0:_D], preferred_element_type=jnp.float32) + be1r[...]
        c = jnp.dot(hn, wl1v[0:_D], preferred_element_type=jnp.float32)
        b = jnp.dot(hn, we1v[_D:2 * _D], preferred_element_type=jnp.float32)
        d = jnp.dot(hn, wl1v[_D:2 * _D], preferred_element_type=jnp.float32)
        srct_ref[...] = jnp.concatenate([a, c], axis=1)
        dstt_ref[...] = jnp.concatenate([b, d], axis=1)

    full = lambda shape: pl.BlockSpec(shape, lambda i: tuple(0 for _ in shape))
    return pl.pallas_call(
        body,
        grid=(_N // _BN,),
        in_specs=[
            pl.BlockSpec((_NC, _BN, _D), lambda i: (0, i, 0)),
            pl.BlockSpec((_BN, _D), lambda i: (i, 0)),
            pl.BlockSpec((_BN, 1), lambda i: (i, 0)),
            full((_D, _D)),
            full((_D, _D)),
            full((_D,)),
            full((_D, _D)),
            full((_D,)),
            full((2 * _D + 5, _D)),
            full((_D,)),
            full((2 * _D + 5, _D)),
        ],
        out_specs=[
            pl.BlockSpec((_BN, 2 * _D), lambda i: (i, 0)),
            pl.BlockSpec((_BN, 2 * _D), lambda i: (i, 0)),
        ],
        out_shape=[
            jax.ShapeDtypeStruct((_N, 2 * _D), jnp.float32),
            jax.ShapeDtypeStruct((_N, 2 * _D), jnp.float32),
        ],
    )(M, h1, dinv, W2s, W2n, b2, Wn, bn, We1, be1, Wl1)


_BE = 2000


def _tc_edge_mlp(Gs, Gd, ea, We1, We2, be2, Wl1, bl1, Wl2, bl2):
    def body(gs_ref, gd_ref, ea_ref, we1, we2, be2r, wl1, bl1r, wl2, bl2r,
             o_ref):
        gs = gs_ref[...]
        gd = gd_ref[...]
        eab = ea_ref[...]
        we1c = we1[...][2 * _D:]
        wl1c = wl1[...][2 * _D:]
        q = (gs[:, :_D] + gd[:, :_D]
             + jnp.dot(eab, we1c, preferred_element_type=jnp.float32))
        g = jnp.tanh(q)
        s = jnp.dot(g, we2[...], preferred_element_type=jnp.float32) + be2r[...]
        w = jax.nn.sigmoid(s)
        p = (gs[:, _D:] + gd[:, _D:]
             + jnp.dot(eab, wl1c, preferred_element_type=jnp.float32))
        t = jnp.maximum(p * w + bl1r[...], 0.0)
        o = jnp.dot(t, wl2[...], preferred_element_type=jnp.float32) + bl2r[...]
        o_ref[...] = o

    full = lambda shape: pl.BlockSpec(shape, lambda i: tuple(0 for _ in shape))
    return pl.pallas_call(
        body,
        grid=(_EP // _BE,),
        in_specs=[
            pl.BlockSpec((_BE, 2 * _D), lambda i: (i, 0)),
            pl.BlockSpec((_BE, 2 * _D), lambda i: (i, 0)),
            pl.BlockSpec((_BE, 5), lambda i: (i, 0)),
            full((2 * _D + 5, _D)),
            full((_D, 1)),
            full((1,)),
            full((2 * _D + 5, _D)),
            full((_D,)),
            full((_D, 1)),
            full((1,)),
        ],
        out_specs=pl.BlockSpec((_BE, 1), lambda i: (i, 0)),
        out_shape=jax.ShapeDtypeStruct((_EP, 1), jnp.float32),
    )(Gs, Gd, ea, We1, We2, be2, Wl1, bl1, Wl2, bl2)


def kernel(x, edge_index, edge_attr, W1_self, W1_neigh, b1, W2_self, W2_neigh,
           b2, Wn, bn, We1, be1, We2, be2, Wl1, bl1, Wl2, bl2):
    srcm = edge_index[0].reshape(_NW, _ERPT)
    dstm = edge_index[1].reshape(_NW, _ERPT)
    src3 = srcm.reshape(_NW, _CHUNKS, _K)
    dst3 = dstm.reshape(_NW, _CHUNKS, _K)
    ea_p = edge_attr
    zeros_d = jnp.zeros((_RCHUNK, _D), jnp.float32)
    zeros1 = jnp.zeros((_ROWS_PT,), jnp.float32)
    ones1 = jnp.ones((_K,), jnp.float32)

    P, degPf = _segsum_sc(_D, with_deg=True)(x, src3, dst3, zeros_d,
                                             zeros1, ones1)
    P = P[:, :_N]
    degP = degPf[:, :_N].reshape(_NC, _N, 1)
    h1, dinv = _tc_layer1(P, degP, x, W1_self, W1_neigh, b1)
    (M,) = _segsum_sc(_D)(h1, src3, dst3, zeros_d)
    M = M[:, :_N]
    srctab, dsttab = _tc_layer2_tables(M, h1, dinv, W2_self, W2_neigh, b2,
                                       Wn, bn, We1, be1, Wl1)
    gs, gd = _edge_gather_sc()(srctab, dsttab, src3, dst3)
    o = _tc_edge_mlp(gs, gd, ea_p, We1, We2, be2, Wl1, bl1, Wl2, bl2)
    return o.reshape(_NW, _EPT)[:, :_ERPT].reshape(_E, 1)
